# baseline (speedup 1.0000x reference)
import functools
import math

import jax
import jax.numpy as jnp
from jax import lax
from jax.experimental import pallas as pl
from jax.experimental.pallas import tpu as pltpu
from jax.experimental.pallas import tpu_sc as plsc

NW = 32


@functools.lru_cache(maxsize=None)
def _mk_gather(e_pad, n_chunks, chunk, d, dtype_name):
    dtype = jnp.dtype(dtype_name)
    per_w = n_chunks * chunk
    mesh = plsc.VectorSubcoreMesh(core_axis_name="c", subcore_axis_name="s")
    if d == 1:
        out_sd = jax.ShapeDtypeStruct((e_pad,), dtype)
        row_scratch = pltpu.VMEM((chunk,), dtype)
    else:
        out_sd = jax.ShapeDtypeStruct((e_pad, d), dtype)
        row_scratch = pltpu.VMEM((chunk, d), dtype)

    @functools.partial(
        pl.kernel,
        mesh=mesh,
        out_type=out_sd,
        scratch_types=[
            pltpu.VMEM((chunk,), jnp.int32),
            row_scratch,
            pltpu.SemaphoreType.DMA,
        ],
    )
    def gather_k(table_hbm, idx_hbm, out_hbm, idx_v, rows_v, sem):
        wid = lax.axis_index("s") * 2 + lax.axis_index("c")
        base = wid * per_w

        def body(j, carry):
            start = base + j * chunk
            pltpu.sync_copy(idx_hbm.at[pl.ds(start, chunk)], idx_v)
            pltpu.async_copy(table_hbm.at[idx_v], rows_v, sem).wait()
            pltpu.sync_copy(rows_v, out_hbm.at[pl.ds(start, chunk)])
            return carry

        lax.fori_loop(0, n_chunks, body, 0)

    return gather_k


def _sc_gather(table, idx):
    e = idx.shape[0]
    if table.ndim == 1:
        d = 1
    else:
        d = table.shape[1]
    chunk = {1: 8192, 4: 2048, 128: 512}.get(d, 512)
    n_chunks = -(-e // (NW * chunk))
    per_w = n_chunks * chunk
    e_pad = per_w * NW
    rows = table.shape[0]
    pad = jnp.arange(e_pad - e, dtype=jnp.int32) % jnp.int32(rows)
    idx_p = jnp.concatenate([idx.astype(jnp.int32), pad])
    k = _mk_gather(e_pad, n_chunks, chunk, d, str(table.dtype))
    out = k(table, idx_p)
    return out[:e]


def _gtab(table, idx):
    rows = table.shape[0]
    idx_c = jnp.clip(idx, 0, rows - 1)
    return _sc_gather(table, idx_c)


def _grow3(table, idx):
    rows = table.shape[0]
    idx_c = jnp.clip(idx, 0, rows - 1)
    cols = [_sc_gather(jnp.reshape(table[:, j], (rows,)), idx_c) for j in range(3)]
    return jnp.stack(cols, axis=1)


def _gcn_v(x, src, dst, W, b):
    N = x.shape[0]
    h = x @ W
    loop = jnp.arange(N, dtype=src.dtype)
    s = jnp.concatenate([src, loop]); d = jnp.concatenate([dst, loop])
    deg = jnp.zeros((N,), jnp.float32).at[d].add(1.0)
    dis = jnp.where(deg > 0, 1.0 / jnp.sqrt(deg), 0.0)
    norm = _gtab(dis, s) * _gtab(dis, d)
    if W.shape[1] >= 8:
        hs = _gtab(h, s)
    else:
        hs = _grow3(h, s)
    out = jnp.zeros((N, W.shape[1]), jnp.float32).at[d].add(hs * norm[:, None])
    return out + b


def _gat_v(x, src, dst, W, att_s, att_d, b):
    N = x.shape[0]
    h = x @ W
    loop = jnp.arange(N, dtype=src.dtype)
    s = jnp.concatenate([src, loop]); d = jnp.concatenate([dst, loop])
    e = _gtab(h @ att_s, s) + _gtab(h @ att_d, d)
    e = jnp.where(e > 0, e, 0.2 * e)
    emax = jnp.full((N,), -jnp.inf, jnp.float32).at[d].max(e)
    a = jnp.exp(e - _gtab(emax, d))
    den = jnp.zeros((N,), jnp.float32).at[d].add(a)
    a = a / _gtab(den, d)
    hs = _gtab(h, s)
    out = jnp.zeros((N, W.shape[1]), jnp.float32).at[d].add(hs * a[:, None])
    return out + b


def _pool_v(h, src, dst, p):
    N = h.shape[0]
    k = int(math.ceil(0.5 * N))
    score = jnp.tanh((h @ p) / jnp.linalg.norm(p))
    _, perm = jax.lax.top_k(score, k)
    x_new = h[perm] * score[perm][:, None]
    mask = jnp.zeros((N,), dtype=bool).at[perm].set(True)
    new_idx = jnp.zeros((N,), dtype=jnp.int32).at[perm].set(jnp.arange(k, dtype=jnp.int32))
    valid = (src < N) & (dst < N)
    mi = jnp.where(mask, new_idx, jnp.int32(-1))
    rs = _gtab(mi, src)
    rd = _gtab(mi, dst)
    em = valid & (rs >= 0) & (rd >= 0)
    s2 = jnp.where(em, rs, jnp.int32(k))
    d2 = jnp.where(em, rd, jnp.int32(k))
    batch = jnp.zeros((k,), jnp.int32)
    return x_new, s2, d2, batch, perm


def kernel(x, edge_index, W_enc0, b_enc0, p0, W_dec0, b_dec0, W_enc1, as1, ad1, b_enc1, p1, W_dec1, b_dec1, W_enc2, as2, ad2, b_enc2, p2, W_dec2, b_dec2):
    src = edge_index[0].astype(jnp.int32); dst = edge_index[1].astype(jnp.int32)
    outputs = []; batches = []
    h = _gcn_v(x, src, dst, W_enc0, b_enc0)
    h, src, dst, batch, _ = _pool_v(h, src, dst, p0)
    h = _gcn_v(h, src, dst, W_dec0, b_dec0)
    outputs.append(h); batches.append(batch)
    h = _gat_v(h, src, dst, W_enc1, as1, ad1, b_enc1)
    h, src, dst, batch, _ = _pool_v(h, src, dst, p1)
    h = _gcn_v(h, src, dst, W_dec1, b_dec1)
    outputs.append(h); batches.append(batch)
    h = _gat_v(h, src, dst, W_enc2, as2, ad2, b_enc2)
    h, src, dst, batch, _ = _pool_v(h, src, dst, p2)
    h = _gcn_v(h, src, dst, W_dec2, b_dec2)
    outputs.append(h); batches.append(batch)
    return (outputs[0], outputs[1], outputs[2], batches[0], batches[1], batches[2])
